# baseline (speedup 1.0000x reference)
import functools

import jax
import jax.numpy as jnp
from jax import lax
from jax.experimental import pallas as pl
from jax.experimental.pallas import tpu as pltpu
from jax.experimental.pallas import tpu_sc as plsc

_NC = 2
_NS = 16
_NW = _NC * _NS
_LANES = 16
_K = 8


def _bag_body(tok, d, rw, ctx_hbm, w_hbm, out_hbm, idx_v, rows_v, out_v,
              *sems):
    nvr = d // _LANES
    wid = lax.axis_index("s") * _NC + lax.axis_index("c")
    base = wid * rw
    pltpu.sync_copy(ctx_hbm.at[pl.ds(wid * (rw // 2), rw // 2)], idx_v)

    def accum(r, buf, half):
        def tok_step(t, acc):
            return tuple(
                acc[j] + rows_v[buf, half * tok + t, pl.ds(_LANES * j, _LANES)]
                for j in range(nvr)
            )
        acc = lax.fori_loop(
            0, tok, tok_step,
            tuple(jnp.zeros((_LANES,), jnp.float32) for _ in range(nvr)),
            unroll=10,
        )
        for j in range(nvr):
            out_v[r, pl.ds(_LANES * j, _LANES)] = acc[j]

    def group_step(g, _):
        descs = []
        for k in range(_K):
            rr = g * _K + k
            descs.append(
                pltpu.async_copy(w_hbm.at[idx_v.at[rr]], rows_v.at[k], sems[k])
            )
        for k in range(_K):
            descs[k].wait()
            rr = g * _K + k
            accum(2 * rr, k, 0)
            accum(2 * rr + 1, k, 1)
        return _

    lax.fori_loop(0, rw // (2 * _K), group_step, 0)
    pltpu.sync_copy(out_v, out_hbm.at[pl.ds(base, rw)])


def _build(batch, tok, vocab, d):
    rw = batch // _NW
    mesh = plsc.VectorSubcoreMesh(core_axis_name="c", subcore_axis_name="s")
    body = functools.partial(_bag_body, tok, d, rw)
    return pl.kernel(
        body,
        out_type=jax.ShapeDtypeStruct((batch, d), jnp.float32),
        mesh=mesh,
        scratch_types=[
            pltpu.VMEM((rw // 2, 2 * tok), jnp.int32),
            pltpu.VMEM((_K, 2 * tok, d), jnp.float32),
            pltpu.VMEM((rw, d), jnp.float32),
        ] + [pltpu.SemaphoreType.DMA] * _K,
        compiler_params=pltpu.CompilerParams(use_tc_tiling_on_sc=False),
    )


def kernel(contexts, weight):
    batch, tok = contexts.shape
    vocab, d = weight.shape
    ids = contexts.astype(jnp.int32).reshape(batch // 2, 2 * tok)
    ds_ = d // 2
    f = _build(batch, tok, vocab, ds_)
    outs = [
        f(ids, lax.slice_in_dim(weight, s * ds_, (s + 1) * ds_, axis=1))
        for s in range(2)
    ]
    return jnp.concatenate(outs, axis=1)


# speedup vs baseline: 2.1988x; 2.1988x over previous
import functools

import jax
import jax.numpy as jnp
from jax import lax
from jax.experimental import pallas as pl
from jax.experimental.pallas import tpu as pltpu
from jax.experimental.pallas import tpu_sc as plsc

_NC = 2
_NS = 16
_NW = _NC * _NS
_LANES = 16
_K = 16


def _bag_body(tok, d, rw, ctx_hbm, w_hbm, out_hbm, idx_v, rows_v, out_v,
              *sems):
    nvr = d // _LANES
    wid = lax.axis_index("s") * _NC + lax.axis_index("c")
    base = wid * rw
    pltpu.sync_copy(ctx_hbm.at[pl.ds(wid * (rw // 2), rw // 2)], idx_v)

    def accum(r, buf, half):
        def tok_step(t, acc):
            return tuple(
                acc[j] + rows_v[buf, half * tok + t, pl.ds(_LANES * j, _LANES)]
                for j in range(nvr)
            )
        acc = lax.fori_loop(
            0, tok, tok_step,
            tuple(jnp.zeros((_LANES,), jnp.float32) for _ in range(nvr)),
            unroll=10,
        )
        for j in range(nvr):
            out_v[r, pl.ds(_LANES * j, _LANES)] = acc[j]

    def group_step(g, _):
        descs = []
        for k in range(_K):
            rr = g * _K + k
            descs.append(
                pltpu.async_copy(w_hbm.at[idx_v.at[rr]], rows_v.at[k], sems[k])
            )
        for k in range(_K):
            descs[k].wait()
            rr = g * _K + k
            accum(2 * rr, k, 0)
            accum(2 * rr + 1, k, 1)
        return _

    lax.fori_loop(0, rw // (2 * _K), group_step, 0)
    pltpu.sync_copy(out_v, out_hbm.at[pl.ds(base, rw)])


def _build(batch, tok, vocab, d):
    rw = batch // _NW
    mesh = plsc.VectorSubcoreMesh(core_axis_name="c", subcore_axis_name="s")
    body = functools.partial(_bag_body, tok, d, rw)
    return pl.kernel(
        body,
        out_type=jax.ShapeDtypeStruct((batch, d), jnp.float32),
        mesh=mesh,
        scratch_types=[
            pltpu.VMEM((rw // 2, 2 * tok), jnp.int32),
            pltpu.VMEM((_K, 2 * tok, d), jnp.float32),
            pltpu.VMEM((rw, d), jnp.float32),
        ] + [pltpu.SemaphoreType.DMA] * _K,
        compiler_params=pltpu.CompilerParams(use_tc_tiling_on_sc=False),
    )


def kernel(contexts, weight):
    batch, tok = contexts.shape
    vocab, d = weight.shape
    f = _build(batch, tok, vocab, d)
    return f(contexts.astype(jnp.int32).reshape(batch // 2, 2 * tok), weight)
